# baseline (device time: 1099512 ns/iter reference)
import jax
import jax.numpy as jnp
from jax import lax
from jax.experimental import pallas as pl
from jax.experimental.pallas import tpu as pltpu


def kernel(x):
    m, n = x.shape
    xb = x.astype(jnp.bfloat16)

    K = 8
    mc = m // K

    def body(x_ref, out_ref, copy_sem, send_sems, recv_sems):
        my_x = lax.axis_index("x")
        my_y = lax.axis_index("y")
        my_z = lax.axis_index("z")
        nbr = (my_x, 1 - my_y, my_z)

        barrier = pltpu.get_barrier_semaphore()
        pl.semaphore_signal(
            barrier, inc=1, device_id=nbr, device_id_type=pl.DeviceIdType.MESH
        )
        pl.semaphore_wait(barrier, 1)

        base = my_y * m
        local = pltpu.make_async_copy(
            x_ref, out_ref.at[pl.ds(base, m), :], copy_sem
        )
        local.start()

        rdmas = []
        for c in range(K):
            rdma = pltpu.make_async_remote_copy(
                src_ref=x_ref.at[pl.ds(c * mc, mc), :],
                dst_ref=out_ref.at[pl.ds(base + c * mc, mc), :],
                send_sem=send_sems.at[c],
                recv_sem=recv_sems.at[c],
                device_id=nbr,
                device_id_type=pl.DeviceIdType.MESH,
            )
            rdma.start()
            rdmas.append(rdma)

        local.wait()
        for rdma in rdmas:
            rdma.wait()

    return pl.pallas_call(
        body,
        out_shape=jax.ShapeDtypeStruct((2 * m, n), jnp.bfloat16),
        in_specs=[pl.BlockSpec(memory_space=pl.ANY)],
        out_specs=pl.BlockSpec(memory_space=pl.ANY),
        scratch_shapes=[
            pltpu.SemaphoreType.DMA,
            pltpu.SemaphoreType.DMA((K,)),
            pltpu.SemaphoreType.DMA((K,)),
        ],
        compiler_params=pltpu.CompilerParams(collective_id=0),
    )(xb)


# device time: 440847 ns/iter; 2.4941x vs baseline; 2.4941x over previous
import jax
import jax.numpy as jnp
from jax import lax
from jax.experimental import pallas as pl
from jax.experimental.pallas import tpu as pltpu


def kernel(x):
    m, n = x.shape
    xb = x.astype(jnp.bfloat16)

    K = 8
    mc = m // K

    def body(x_ref, out_ref, vbufs, ld_sems, st_sems, send_sems, recv_sems):
        my_x = lax.axis_index("x")
        my_y = lax.axis_index("y")
        my_z = lax.axis_index("z")
        nbr = (my_x, 1 - my_y, my_z)

        barrier = pltpu.get_barrier_semaphore()
        pl.semaphore_signal(
            barrier, inc=1, device_id=nbr, device_id_type=pl.DeviceIdType.MESH
        )
        pl.semaphore_wait(barrier, 1)

        base = my_y * m

        rdmas = []
        for c in range(K):
            rdma = pltpu.make_async_remote_copy(
                src_ref=x_ref.at[pl.ds(c * mc, mc), :],
                dst_ref=out_ref.at[pl.ds(base + c * mc, mc), :],
                send_sem=send_sems.at[c],
                recv_sem=recv_sems.at[c],
                device_id=nbr,
                device_id_type=pl.DeviceIdType.MESH,
            )
            rdma.start()
            rdmas.append(rdma)

        stores = [None] * K
        for c in range(K):
            sl = c % 2
            if c >= 2:
                stores[c - 2].wait()
            ld = pltpu.make_async_copy(
                x_ref.at[pl.ds(c * mc, mc), :], vbufs.at[sl], ld_sems.at[sl]
            )
            ld.start()
            ld.wait()
            st = pltpu.make_async_copy(
                vbufs.at[sl],
                out_ref.at[pl.ds(base + c * mc, mc), :],
                st_sems.at[sl],
            )
            st.start()
            stores[c] = st
        stores[K - 2].wait()
        stores[K - 1].wait()

        for rdma in rdmas:
            rdma.wait()

    return pl.pallas_call(
        body,
        out_shape=jax.ShapeDtypeStruct((2 * m, n), jnp.bfloat16),
        in_specs=[pl.BlockSpec(memory_space=pl.ANY)],
        out_specs=pl.BlockSpec(memory_space=pl.ANY),
        scratch_shapes=[
            pltpu.VMEM((2, mc, n), jnp.bfloat16),
            pltpu.SemaphoreType.DMA((2,)),
            pltpu.SemaphoreType.DMA((2,)),
            pltpu.SemaphoreType.DMA((K,)),
            pltpu.SemaphoreType.DMA((K,)),
        ],
        compiler_params=pltpu.CompilerParams(collective_id=0),
    )(xb)


# device time: 410195 ns/iter; 2.6805x vs baseline; 1.0747x over previous
import jax
import jax.numpy as jnp
from jax import lax
from jax.experimental import pallas as pl
from jax.experimental.pallas import tpu as pltpu


def kernel(x):
    m, n = x.shape
    K = 8
    mc = m // K

    def body(x_ref, out_ref, f32_bufs, bf_bufs, ld_sems, st_sems,
             send_sems, recv_sems):
        my_x = lax.axis_index("x")
        my_y = lax.axis_index("y")
        my_z = lax.axis_index("z")
        nbr = (my_x, 1 - my_y, my_z)

        barrier = pltpu.get_barrier_semaphore()
        pl.semaphore_signal(
            barrier, inc=1, device_id=nbr, device_id_type=pl.DeviceIdType.MESH
        )
        pl.semaphore_wait(barrier, 1)

        base = my_y * m

        def load(c):
            cp = pltpu.make_async_copy(
                x_ref.at[pl.ds(c * mc, mc), :],
                f32_bufs.at[c % 2],
                ld_sems.at[c % 2],
            )
            cp.start()
            return cp

        loads = [None] * K
        rdmas = [None] * K
        stores = [None] * K
        loads[0] = load(0)
        for c in range(K):
            sl = c % 2
            if c + 1 < K:
                loads[c + 1] = load(c + 1)
            loads[c].wait()
            if c >= 2:
                rdmas[c - 2].wait_send()
                stores[c - 2].wait()
            bf_bufs[sl, :, :] = f32_bufs[sl, :, :].astype(jnp.bfloat16)
            rdmas[c] = pltpu.make_async_remote_copy(
                src_ref=bf_bufs.at[sl],
                dst_ref=out_ref.at[pl.ds(base + c * mc, mc), :],
                send_sem=send_sems.at[c],
                recv_sem=recv_sems.at[c],
                device_id=nbr,
                device_id_type=pl.DeviceIdType.MESH,
            )
            rdmas[c].start()
            stores[c] = pltpu.make_async_copy(
                bf_bufs.at[sl],
                out_ref.at[pl.ds(base + c * mc, mc), :],
                st_sems.at[sl],
            )
            stores[c].start()

        rdmas[K - 2].wait_send()
        rdmas[K - 1].wait_send()
        stores[K - 2].wait()
        stores[K - 1].wait()
        for c in range(K):
            rdmas[c].wait_recv()

    return pl.pallas_call(
        body,
        out_shape=jax.ShapeDtypeStruct((2 * m, n), jnp.bfloat16),
        in_specs=[pl.BlockSpec(memory_space=pl.ANY)],
        out_specs=pl.BlockSpec(memory_space=pl.ANY),
        scratch_shapes=[
            pltpu.VMEM((2, mc, n), jnp.float32),
            pltpu.VMEM((2, mc, n), jnp.bfloat16),
            pltpu.SemaphoreType.DMA((2,)),
            pltpu.SemaphoreType.DMA((2,)),
            pltpu.SemaphoreType.DMA((K,)),
            pltpu.SemaphoreType.DMA((K,)),
        ],
        compiler_params=pltpu.CompilerParams(collective_id=0),
    )(x)


# device time: 408868 ns/iter; 2.6892x vs baseline; 1.0032x over previous
import jax
import jax.numpy as jnp
from jax import lax
from jax.experimental import pallas as pl
from jax.experimental.pallas import tpu as pltpu


def kernel(x):
    m, n = x.shape
    K = 16
    mc = m // K

    def body(x_ref, out_ref, f32_bufs, bf_bufs, ld_sems, st_sems,
             send_sems, recv_sems):
        my_x = lax.axis_index("x")
        my_y = lax.axis_index("y")
        my_z = lax.axis_index("z")
        nbr = (my_x, 1 - my_y, my_z)

        barrier = pltpu.get_barrier_semaphore()
        pl.semaphore_signal(
            barrier, inc=1, device_id=nbr, device_id_type=pl.DeviceIdType.MESH
        )
        pl.semaphore_wait(barrier, 1)

        base = my_y * m

        def load(c):
            cp = pltpu.make_async_copy(
                x_ref.at[pl.ds(c * mc, mc), :],
                f32_bufs.at[c % 2],
                ld_sems.at[c % 2],
            )
            cp.start()
            return cp

        loads = [None] * K
        rdmas = [None] * K
        stores = [None] * K
        loads[0] = load(0)
        for c in range(K):
            sl = c % 2
            if c + 1 < K:
                loads[c + 1] = load(c + 1)
            loads[c].wait()
            if c >= 2:
                rdmas[c - 2].wait_send()
                stores[c - 2].wait()
            bf_bufs[sl, :, :] = f32_bufs[sl, :, :].astype(jnp.bfloat16)
            rdmas[c] = pltpu.make_async_remote_copy(
                src_ref=bf_bufs.at[sl],
                dst_ref=out_ref.at[pl.ds(base + c * mc, mc), :],
                send_sem=send_sems.at[c],
                recv_sem=recv_sems.at[c],
                device_id=nbr,
                device_id_type=pl.DeviceIdType.MESH,
            )
            rdmas[c].start()
            stores[c] = pltpu.make_async_copy(
                bf_bufs.at[sl],
                out_ref.at[pl.ds(base + c * mc, mc), :],
                st_sems.at[sl],
            )
            stores[c].start()

        rdmas[K - 2].wait_send()
        rdmas[K - 1].wait_send()
        stores[K - 2].wait()
        stores[K - 1].wait()
        for c in range(K):
            rdmas[c].wait_recv()

    return pl.pallas_call(
        body,
        out_shape=jax.ShapeDtypeStruct((2 * m, n), jnp.bfloat16),
        in_specs=[pl.BlockSpec(memory_space=pl.ANY)],
        out_specs=pl.BlockSpec(memory_space=pl.ANY),
        scratch_shapes=[
            pltpu.VMEM((2, mc, n), jnp.float32),
            pltpu.VMEM((2, mc, n), jnp.bfloat16),
            pltpu.SemaphoreType.DMA((2,)),
            pltpu.SemaphoreType.DMA((2,)),
            pltpu.SemaphoreType.DMA((K,)),
            pltpu.SemaphoreType.DMA((K,)),
        ],
        compiler_params=pltpu.CompilerParams(collective_id=0),
    )(x)
